# baseline (speedup 1.0000x reference)
import jax
import jax.numpy as jnp
from jax.experimental import pallas as pl
from jax.experimental.pallas import tpu as pltpu

_T, _D, _F, _G, _E = 2048, 768, 3072, 3, 8
_GE = _G * _E
_C = 2
_EC = _GE // _C
_FC = 1024
_NF = _F // _FC


def _moe_body(x_ref, wg_ref, bg_ref, wr_ref, br_ref, b2r_ref,
              w1_ref, b1_ref, w2_ref, out_ref, w_scr):
    c = pl.program_id(0)
    e = pl.program_id(1)
    f = pl.program_id(2)

    @pl.when((e == 0) & (f == 0))
    def _init():
        x = x_ref[...]
        gl = jnp.dot(x, wg_ref[...], preferred_element_type=jnp.float32)
        gl = gl + bg_ref[...]
        gl = gl - jnp.max(gl, axis=1, keepdims=True)
        gp = jnp.exp(gl)
        gp = gp / jnp.sum(gp, axis=1, keepdims=True)
        el = jnp.dot(x, wr_ref[...], preferred_element_type=jnp.float32)
        el = el + br_ref[...]
        cols = []
        for g in range(_G):
            sl = el[:, g * _E:(g + 1) * _E]
            sl = sl - jnp.max(sl, axis=1, keepdims=True)
            p = jnp.exp(sl)
            p = p / jnp.sum(p, axis=1, keepdims=True)
            cols.append(p * gp[:, g:g + 1])
        w = jnp.concatenate(cols, axis=1)
        w_scr[...] = w
        lane = jax.lax.broadcasted_iota(jnp.int32, (_T, _GE), 1)
        wc = jnp.where(lane // _EC == c, w, 0.0)
        out_ref[0] = jnp.dot(wc, b2r_ref[...],
                             preferred_element_type=jnp.float32)

    x = x_ref[...]
    w1 = w1_ref[0, 0].astype(jnp.bfloat16)
    t = jnp.dot(x, w1, preferred_element_type=jnp.float32)
    t = t + b1_ref[0, 0]
    t = 0.5 * t * (1.0 + jax.lax.erf(t * 0.7071067811865476))
    lane = jax.lax.broadcasted_iota(jnp.int32, (_T, _GE), 1)
    wsel = jnp.sum(jnp.where(lane == c * _EC + e, w_scr[...], 0.0),
                   axis=1, keepdims=True)
    t = (t * wsel).astype(jnp.bfloat16)
    w2 = w2_ref[0, 0].astype(jnp.bfloat16)
    out_ref[0] += jnp.dot(t, w2, preferred_element_type=jnp.float32)


def kernel(h_fused, Wg, bg, Wr, br, W1, b1, W2, b2):
    x_bf = h_fused.astype(jnp.bfloat16)
    wg_bf = Wg.astype(jnp.bfloat16)
    wr_bf = Wr.transpose(1, 0, 2).reshape(_D, _GE).astype(jnp.bfloat16)
    bg2 = bg.reshape(1, _G)
    br2 = br.reshape(1, _GE)
    w1r = W1.reshape(_C, _EC, _D, _F)
    b1r = b1.reshape(_C, _EC, 1, _F)
    w2r = W2.reshape(_C, _EC, _F, _D)
    b2r = b2.reshape(_GE, _D)

    partials = pl.pallas_call(
        _moe_body,
        grid=(_C, _EC, _NF),
        in_specs=[
            pl.BlockSpec((_T, _D), lambda c, e, f: (0, 0)),
            pl.BlockSpec((_D, _G), lambda c, e, f: (0, 0)),
            pl.BlockSpec((1, _G), lambda c, e, f: (0, 0)),
            pl.BlockSpec((_D, _GE), lambda c, e, f: (0, 0)),
            pl.BlockSpec((1, _GE), lambda c, e, f: (0, 0)),
            pl.BlockSpec((_GE, _D), lambda c, e, f: (0, 0)),
            pl.BlockSpec((1, 1, _D, _FC), lambda c, e, f: (c, e, 0, f)),
            pl.BlockSpec((1, 1, 1, _FC), lambda c, e, f: (c, e, 0, f)),
            pl.BlockSpec((1, 1, _FC, _D), lambda c, e, f: (c, e, f, 0)),
        ],
        out_specs=pl.BlockSpec((1, _T, _D), lambda c, e, f: (c, 0, 0)),
        out_shape=jax.ShapeDtypeStruct((_C, _T, _D), jnp.float32),
        scratch_shapes=[pltpu.VMEM((_T, _GE), jnp.float32)],
        compiler_params=pltpu.CompilerParams(
            dimension_semantics=("parallel", "arbitrary", "arbitrary")),
    )(x_bf, wg_bf, bg2, wr_bf, br2, b2r, w1r, b1r, w2r)
    return partials[0] + partials[1]


# speedup vs baseline: 1.0741x; 1.0741x over previous
import jax
import jax.numpy as jnp
from jax.experimental import pallas as pl
from jax.experimental.pallas import tpu as pltpu

_T, _D, _F, _G, _E = 2048, 768, 3072, 3, 8
_GE = _G * _E
_FC = 1024
_NF = _F // _FC


def _moe_body(x_ref, wg_ref, bg_ref, wr_ref, br_ref, b2r_ref,
              w1_ref, b1_ref, w2_ref, out_ref, w_scr):
    e = pl.program_id(0)
    f = pl.program_id(1)

    @pl.when((e == 0) & (f == 0))
    def _init():
        x = x_ref[...]
        gl = jnp.dot(x, wg_ref[...], preferred_element_type=jnp.float32)
        gl = gl + bg_ref[...]
        gl = gl - jnp.max(gl, axis=1, keepdims=True)
        gp = jnp.exp(gl)
        gp = gp / jnp.sum(gp, axis=1, keepdims=True)
        el = jnp.dot(x, wr_ref[...], preferred_element_type=jnp.float32)
        el = el + br_ref[...]
        cols = []
        for g in range(_G):
            sl = el[:, g * _E:(g + 1) * _E]
            sl = sl - jnp.max(sl, axis=1, keepdims=True)
            p = jnp.exp(sl)
            p = p / jnp.sum(p, axis=1, keepdims=True)
            cols.append(p * gp[:, g:g + 1])
        w = jnp.concatenate(cols, axis=1)
        w_scr[...] = w
        out_ref[...] = jnp.dot(w, b2r_ref[...],
                               preferred_element_type=jnp.float32)

    x = x_ref[...]
    w1 = w1_ref[0].astype(jnp.bfloat16)
    t = jnp.dot(x, w1, preferred_element_type=jnp.float32)
    t = t + b1_ref[0]
    lane = jax.lax.broadcasted_iota(jnp.int32, (_T, _GE), 1)
    wsel = jnp.sum(jnp.where(lane == e, w_scr[...], 0.0),
                   axis=1, keepdims=True)
    u = (t * 0.7071067811865476).astype(jnp.bfloat16)
    v = jax.lax.erf(u)
    a = (t * (wsel * 0.5)).astype(jnp.bfloat16)
    h = a + a * v
    w2 = w2_ref[0].astype(jnp.bfloat16)
    out_ref[...] += jnp.dot(h, w2, preferred_element_type=jnp.float32)


def kernel(h_fused, Wg, bg, Wr, br, W1, b1, W2, b2):
    x_bf = h_fused.astype(jnp.bfloat16)
    wg_bf = Wg.astype(jnp.bfloat16)
    wr_bf = Wr.transpose(1, 0, 2).reshape(_D, _GE).astype(jnp.bfloat16)
    bg2 = bg.reshape(1, _G)
    br2 = br.reshape(1, _GE)
    w1r = W1.reshape(_GE, _D, _F)
    b1r = b1.reshape(_GE, 1, _F)
    w2r = W2.reshape(_GE, _F, _D)
    b2r = b2.reshape(_GE, _D)

    out = pl.pallas_call(
        _moe_body,
        grid=(_GE, _NF),
        in_specs=[
            pl.BlockSpec((_T, _D), lambda e, f: (0, 0)),
            pl.BlockSpec((_D, _G), lambda e, f: (0, 0)),
            pl.BlockSpec((1, _G), lambda e, f: (0, 0)),
            pl.BlockSpec((_D, _GE), lambda e, f: (0, 0)),
            pl.BlockSpec((1, _GE), lambda e, f: (0, 0)),
            pl.BlockSpec((_GE, _D), lambda e, f: (0, 0)),
            pl.BlockSpec((1, _D, _FC), lambda e, f: (e, 0, f)),
            pl.BlockSpec((1, 1, _FC), lambda e, f: (e, 0, f)),
            pl.BlockSpec((1, _FC, _D), lambda e, f: (e, f, 0)),
        ],
        out_specs=pl.BlockSpec((_T, _D), lambda e, f: (0, 0)),
        out_shape=jax.ShapeDtypeStruct((_T, _D), jnp.float32),
        scratch_shapes=[pltpu.VMEM((_T, _GE), jnp.float32)],
    )(x_bf, wg_bf, bg2, wr_bf, br2, b2r, w1r, b1r, w2r)
    return out


# speedup vs baseline: 1.1111x; 1.0345x over previous
import jax
import jax.numpy as jnp
from jax.experimental import pallas as pl
from jax.experimental.pallas import tpu as pltpu

_T, _D, _F, _G, _E = 2048, 768, 3072, 3, 8
_GE = _G * _E
_FC = 1024
_NF = _F // _FC


def _moe_body(x_ref, wg_ref, bg_ref, wr_ref, br_ref, b2r_ref,
              w1_ref, b1_ref, w2_ref, out_ref, w_scr):
    e = pl.program_id(0)
    f = pl.program_id(1)

    @pl.when((e == 0) & (f == 0))
    def _init():
        x = x_ref[...]
        gl = jnp.dot(x, wg_ref[...], preferred_element_type=jnp.float32)
        gl = gl + bg_ref[...]
        gl = gl - jnp.max(gl, axis=1, keepdims=True)
        gp = jnp.exp(gl)
        gp = gp / jnp.sum(gp, axis=1, keepdims=True)
        el = jnp.dot(x, wr_ref[...], preferred_element_type=jnp.float32)
        el = el + br_ref[...]
        cols = []
        for g in range(_G):
            sl = el[:, g * _E:(g + 1) * _E]
            sl = sl - jnp.max(sl, axis=1, keepdims=True)
            p = jnp.exp(sl)
            p = p / jnp.sum(p, axis=1, keepdims=True)
            cols.append(p * gp[:, g:g + 1])
        w = jnp.concatenate(cols, axis=1)
        w_scr[...] = w * 0.5
        out_ref[...] = jnp.dot(w, b2r_ref[...],
                               preferred_element_type=jnp.float32)


    x = x_ref[...]
    w1 = w1_ref[0].astype(jnp.bfloat16)
    t = jnp.dot(x, w1, preferred_element_type=jnp.float32)
    t = t + b1_ref[0]
    lane = jax.lax.broadcasted_iota(jnp.int32, (_T, _GE), 1)
    wselh = jnp.sum(jnp.where(lane == e, w_scr[...], 0.0),
                    axis=1, keepdims=True).astype(jnp.bfloat16)
    t_bf = t.astype(jnp.bfloat16)
    v = jax.lax.erf(t_bf * jnp.bfloat16(0.7071067811865476))
    a = t_bf * wselh
    h = a + a * v
    w2 = w2_ref[0].astype(jnp.bfloat16)
    out_ref[...] += jnp.dot(h, w2, preferred_element_type=jnp.float32)


def kernel(h_fused, Wg, bg, Wr, br, W1, b1, W2, b2):
    x_bf = h_fused.astype(jnp.bfloat16)
    wg_bf = Wg.astype(jnp.bfloat16)
    wr_bf = Wr.transpose(1, 0, 2).reshape(_D, _GE).astype(jnp.bfloat16)
    bg2 = bg.reshape(1, _G)
    br2 = br.reshape(1, _GE)
    w1r = W1.reshape(_GE, _D, _F)
    b1r = b1.reshape(_GE, 1, _F)
    w2r = W2.reshape(_GE, _F, _D)
    b2r = b2.reshape(_GE, _D)

    out = pl.pallas_call(
        _moe_body,
        grid=(_GE, _NF),
        in_specs=[
            pl.BlockSpec((_T, _D), lambda e, f: (0, 0)),
            pl.BlockSpec((_D, _G), lambda e, f: (0, 0)),
            pl.BlockSpec((1, _G), lambda e, f: (0, 0)),
            pl.BlockSpec((_D, _GE), lambda e, f: (0, 0)),
            pl.BlockSpec((1, _GE), lambda e, f: (0, 0)),
            pl.BlockSpec((_GE, _D), lambda e, f: (0, 0)),
            pl.BlockSpec((1, _D, _FC), lambda e, f: (e, 0, f)),
            pl.BlockSpec((1, 1, _FC), lambda e, f: (e, 0, f)),
            pl.BlockSpec((1, _FC, _D), lambda e, f: (e, f, 0)),
        ],
        out_specs=pl.BlockSpec((_T, _D), lambda e, f: (0, 0)),
        out_shape=jax.ShapeDtypeStruct((_T, _D), jnp.float32),
        scratch_shapes=[pltpu.VMEM((_T, _GE), jnp.float32)],
    )(x_bf, wg_bf, bg2, wr_bf, br2, b2r, w1r, b1r, w2r)
    return out
